# Initial kernel scaffold; baseline (speedup 1.0000x reference)
import jax
import jax.numpy as jnp
from jax.experimental import pallas as pl


def kernel(embeddings, WQ_prog, WK_prog, WV_op, WV_arg, WQ_stack, WK_stack, WV_stack):
    raise NotImplementedError("write your pallas kernel here")


# speedup vs baseline: 1.3027x; 1.3027x over previous
import functools

import jax
import jax.numpy as jnp
from jax import lax
from jax.experimental import pallas as pl
from jax.experimental.pallas import tpu as pltpu
from jax.experimental.pallas import tpu_sc as plsc

_T = 8192
_QB = 256
_GRID = _T // _QB

_NC = 2
_NS = 16
_NW = _NC * _NS
_CHUNK = _T // _NW
_L = 16


def _argmax_body(emb_ref, embt_ref, wqpt_ref, wkp_ref, wvot_ref, wvat_ref,
                 wqst_ref, wks_ref, wvst_ref,
                 bp_ref, bs_ref, vo_ref, va_ref, vs_ref):
    pid = pl.program_id(0)
    qrows = emb_ref[pl.ds(pid * _QB, _QB), :]
    embt = embt_ref[...]

    iota = lax.broadcasted_iota(jnp.int32, (_QB, _T), 1)

    def head(wqt_ref, wk_ref):
        q = jnp.dot(qrows, wqt_ref[...])
        k = jnp.dot(wk_ref[...], embt)
        s = jnp.dot(q, k)
        m = jnp.max(s, axis=1, keepdims=True)
        idx = jnp.min(jnp.where(s == m, iota, _T), axis=1, keepdims=True)
        return idx

    rows = pl.ds(pid * _QB, _QB)
    bp_ref[rows, :] = head(wqpt_ref, wkp_ref)
    bs_ref[rows, :] = head(wqst_ref, wks_ref)
    vo_ref[rows, :] = jnp.dot(qrows, wvot_ref[...])
    va_ref[rows, :] = jnp.dot(qrows, wvat_ref[...])
    vs_ref[rows, :] = jnp.dot(qrows, wvst_ref[...])


def _tc_argmax(emb, embt, wqpt, wkp, wvot, wvat, wqst, wks, wvst):
    full = lambda a: pl.BlockSpec(a.shape, lambda i: (0,) * a.ndim)
    out_sd = jax.ShapeDtypeStruct((_T, 1), jnp.int32)
    out_sf = jax.ShapeDtypeStruct((_T, 1), jnp.float32)
    return pl.pallas_call(
        _argmax_body,
        grid=(_GRID,),
        in_specs=[full(a) for a in (emb, embt, wqpt, wkp, wvot, wvat, wqst, wks, wvst)],
        out_specs=[pl.BlockSpec((_T, 1), lambda i: (0, 0))] * 5,
        out_shape=[out_sd, out_sd, out_sf, out_sf, out_sf],
    )(emb, embt, wqpt, wkp, wvot, wvat, wqst, wks, wvst)


def _sc_gather_body(vop_hbm, varg_hbm, vstk_hbm, ip_hbm, is_hbm,
                    oop_hbm, oarg_hbm, oval_hbm,
                    vop_v, varg_v, vstk_v, ip_v, is_v, oop_v, oarg_v, oval_v):
    wid = lax.axis_index("s") * _NC + lax.axis_index("c")
    base = wid * _CHUNK
    pltpu.sync_copy(vop_hbm, vop_v)
    pltpu.sync_copy(varg_hbm, varg_v)
    pltpu.sync_copy(vstk_hbm, vstk_v)
    pltpu.sync_copy(ip_hbm.at[pl.ds(base, _CHUNK)], ip_v)
    pltpu.sync_copy(is_hbm.at[pl.ds(base, _CHUNK)], is_v)
    for j in range(_CHUNK // _L):
        sl = pl.ds(j * _L, _L)
        idxp = ip_v[sl]
        oop_v[sl] = plsc.load_gather(vop_v, [idxp])
        oarg_v[sl] = plsc.load_gather(varg_v, [idxp])
        idxs = is_v[sl]
        oval_v[sl] = plsc.load_gather(vstk_v, [idxs])
    pltpu.sync_copy(oop_v, oop_hbm.at[pl.ds(base, _CHUNK)])
    pltpu.sync_copy(oarg_v, oarg_hbm.at[pl.ds(base, _CHUNK)])
    pltpu.sync_copy(oval_v, oval_hbm.at[pl.ds(base, _CHUNK)])


@functools.cache
def _sc_gather():
    return functools.partial(
        pl.kernel,
        mesh=plsc.VectorSubcoreMesh(core_axis_name="c", subcore_axis_name="s"),
        compiler_params=pltpu.CompilerParams(
            use_tc_tiling_on_sc=False, needs_layout_passes=False),
        out_type=[jax.ShapeDtypeStruct((_T,), jnp.float32)] * 3,
        scratch_types=[
            pltpu.VMEM((_T,), jnp.float32),
            pltpu.VMEM((_T,), jnp.float32),
            pltpu.VMEM((_T,), jnp.float32),
            pltpu.VMEM((_CHUNK,), jnp.int32),
            pltpu.VMEM((_CHUNK,), jnp.int32),
            pltpu.VMEM((_CHUNK,), jnp.float32),
            pltpu.VMEM((_CHUNK,), jnp.float32),
            pltpu.VMEM((_CHUNK,), jnp.float32),
        ],
    )(_sc_gather_body)


def kernel(embeddings, WQ_prog, WK_prog, WV_op, WV_arg, WQ_stack, WK_stack, WV_stack):
    emb = embeddings
    bp, bs, vo, va, vs = _tc_argmax(
        emb, emb.T, WQ_prog.T, WK_prog, WV_op.T, WV_arg.T,
        WQ_stack.T, WK_stack, WV_stack.T)
    fetched_ops, fetched_args, fetched_vals = _sc_gather()(
        vo.reshape(_T), va.reshape(_T), vs.reshape(_T),
        bp.reshape(_T), bs.reshape(_T))
    return (fetched_ops, fetched_args, fetched_vals)


# speedup vs baseline: 2.1611x; 1.6590x over previous
import functools

import numpy as np
import jax
import jax.numpy as jnp
from jax import lax
from jax.experimental import pallas as pl
from jax.experimental.pallas import tpu as pltpu
from jax.experimental.pallas import tpu_sc as plsc

_T = 8192
_M = 256
_MC = 64
_C = 128
_COEF = 2.0 * np.pi / _M + 1e-4
_SENT = 2**30

_ANG = 2.0 * np.pi * np.arange(_M) / _M
_DIRS = np.stack([np.cos(_ANG), np.sin(_ANG)]).astype(np.float32)

_NC = 2
_NS = 16
_L = 16
_NTAB = 8


def _extract_candidates(mask_row, iota128):
    rows = [
        jnp.where(mask_row[0:1, r * 128:(r + 1) * 128] > 0.0,
                  iota128 + r * 128, _SENT)
        for r in range(_T // 128)
    ]
    mi = jnp.concatenate(rows, axis=0)
    out = []
    c0 = None
    for _ in range(_C):
        cur = jnp.min(jnp.min(mi, axis=1, keepdims=True), axis=0,
                      keepdims=True)
        if c0 is None:
            c0 = cur
        out.append(jnp.where(cur < _SENT, cur, c0))
        mi = jnp.where(mi == cur, _SENT, mi)
    return jnp.concatenate(out, axis=1)


def _a_body(emb_ref, embt_ref, wpack_ref, wdpt_ref, wdst_ref,
            cp_ref, cs_ref, wide_ref):
    emb = emb_ref[...]
    embt = embt_ref[...]
    iota128 = lax.broadcasted_iota(jnp.int32, (1, 128), 1)

    wide = jnp.dot(emb, wpack_ref[...])
    wide_ref[...] = wide

    def head(wdt_ref, kcol, c_ref):
        k2 = wide[:, kcol:kcol + 2]
        r2 = jnp.max(k2[:, 0:1] ** 2 + k2[:, 1:2] ** 2, axis=0, keepdims=True)
        delta = jnp.sqrt(r2) * _COEF
        mask_row = None
        for c in range(_M // _MC):
            pc = jnp.dot(wdt_ref[c * _MC:(c + 1) * _MC, :], embt)
            mxc = jnp.max(pc, axis=1, keepdims=True)
            hit = (pc >= mxc - delta).astype(jnp.float32)
            part = jnp.max(hit, axis=0, keepdims=True)
            mask_row = part if mask_row is None else jnp.maximum(mask_row, part)
        c_ref[...] = _extract_candidates(mask_row, iota128)

    head(wdpt_ref, 4, cp_ref)
    head(wdst_ref, 6, cs_ref)


def _stage_a(emb, embt, wpack, wdpt, wdst):
    sd = jax.ShapeDtypeStruct
    return pl.pallas_call(
        _a_body,
        out_shape=[
            sd((1, _C), jnp.int32), sd((1, _C), jnp.int32),
            sd((_T, 16), jnp.float32),
        ],
    )(emb, embt, wpack, wdpt, wdst)


def _b_body(tab_hbm, idx_hbm, out_hbm, tab_v, idx_v, out_v):
    wid = lax.axis_index("s") * _NC + lax.axis_index("c")
    for rep in range(2):
        task = wid * 2 + rep
        tbase = (task // 8) * _T
        pltpu.sync_copy(tab_hbm.at[pl.ds(tbase, _T)], tab_v)
        pltpu.sync_copy(idx_hbm.at[pl.ds(task * _L, _L)], idx_v)
        out_v[...] = plsc.load_gather(tab_v, [idx_v[...]])
        pltpu.sync_copy(out_v, out_hbm.at[pl.ds(task * _L, _L)])


@functools.cache
def _stage_b():
    return functools.partial(
        pl.kernel,
        mesh=plsc.VectorSubcoreMesh(core_axis_name="c", subcore_axis_name="s"),
        compiler_params=pltpu.CompilerParams(
            use_tc_tiling_on_sc=False, needs_layout_passes=False),
        out_type=jax.ShapeDtypeStruct((_NTAB * _C,), jnp.float32),
        scratch_types=[
            pltpu.VMEM((_T,), jnp.float32),
            pltpu.VMEM((_L,), jnp.int32),
            pltpu.VMEM((_L,), jnp.float32),
        ],
    )(_b_body)


def _c_body(wide_ref, kcp_ref, kcs_ref, pvp_ref, pvs_ref, foa_ref, fv_ref):
    iota = lax.broadcasted_iota(jnp.int32, (_T, _C), 1)

    def head(qcol, kc_ref):
        q2 = wide_ref[:, qcol:qcol + 2]
        s = jnp.dot(q2, kc_ref[...])
        m = jnp.max(s, axis=1, keepdims=True)
        pos = jnp.min(jnp.where(s == m, iota, _C), axis=1, keepdims=True)
        return (iota == pos).astype(jnp.float32)

    foa_ref[...] = jnp.dot(head(0, kcp_ref), pvp_ref[...])
    fv_ref[...] = jnp.dot(head(2, kcs_ref), pvs_ref[...])


def _stage_c(wide, kcp, kcs, pvp, pvs):
    sd = jax.ShapeDtypeStruct
    return pl.pallas_call(
        _c_body,
        out_shape=[sd((_T, 2), jnp.float32), sd((_T, 1), jnp.float32)],
    )(wide, kcp, kcs, pvp, pvs)


def kernel(embeddings, WQ_prog, WK_prog, WV_op, WV_arg, WQ_stack, WK_stack, WV_stack):
    dirs = jnp.asarray(_DIRS)
    wdpt = (WK_prog.T @ dirs).T
    wdst = (WK_stack.T @ dirs).T
    wpack = jnp.concatenate([
        WQ_prog.T, WQ_stack.T, WK_prog.T, WK_stack.T,
        WV_op.T, WV_arg.T, WV_stack.T,
        jnp.zeros((36, 5), jnp.float32)], axis=1)

    cp, cs, wide = _stage_a(embeddings, embeddings.T, wpack, wdpt, wdst)

    tab = jnp.concatenate([
        wide[:, 4], wide[:, 5], wide[:, 8], wide[:, 9],
        wide[:, 6], wide[:, 7], wide[:, 10], wide[:, 4]])
    cpf = cp.reshape(_C)
    csf = cs.reshape(_C)
    idx = jnp.concatenate([cpf, cpf, cpf, cpf, csf, csf, csf, cpf])
    out_all = _stage_b()(tab, idx)
    g = out_all.reshape(_NTAB, _C)

    kcp = g[0:2]
    kcs = g[4:6]
    pvp = jnp.stack([g[2], g[3]], axis=1)
    pvs = g[6].reshape(_C, 1)
    foa, fv = _stage_c(wide, kcp, kcs, pvp, pvs)
    return (foa[:, 0], foa[:, 1], fv.reshape(_T))


# speedup vs baseline: 3.2264x; 1.4930x over previous
import functools

import numpy as np
import jax
import jax.numpy as jnp
from jax import lax
from jax.experimental import pallas as pl
from jax.experimental.pallas import tpu as pltpu
from jax.experimental.pallas import tpu_sc as plsc

_T = 8192
_M = 256
_MC = 64
_C = 64
_COEF = 2.0 * np.pi / _M + 1e-4
_SENT = 2**30

_ANG = 2.0 * np.pi * np.arange(_M) / _M
_DIRS = np.stack([np.cos(_ANG), np.sin(_ANG)]).astype(np.float32)

_NC = 2
_NS = 16
_L = 16
_NTAB = 8


def _a_body(emb_ref, embt_ref, wq4_ref, wtab_ref, wdpt_ref, wdst_ref,
            tabs_ref, idx8_ref, qw_ref):
    emb = emb_ref[...]
    embt = embt_ref[...]
    iota128 = lax.broadcasted_iota(jnp.int32, (1, 128), 1)

    qw_ref[...] = jnp.dot(emb, wq4_ref[...])
    tabs = jnp.dot(wtab_ref[...], embt)
    tabs_ref[...] = tabs

    def mask_of(wdt_ref, krow):
        k0 = tabs[krow:krow + 1, :]
        k1 = tabs[krow + 1:krow + 2, :]
        r2 = jnp.max(k0 * k0 + k1 * k1, axis=1, keepdims=True)
        delta = jnp.sqrt(r2) * _COEF
        mask_row = None
        for c in range(_M // _MC):
            pc = jnp.dot(wdt_ref[c * _MC:(c + 1) * _MC, :], embt)
            mxc = jnp.max(pc, axis=1, keepdims=True)
            hit = (pc >= mxc - delta).astype(jnp.float32)
            part = jnp.max(hit, axis=0, keepdims=True)
            mask_row = part if mask_row is None else jnp.maximum(mask_row, part)
        rows = [
            jnp.where(mask_row[0:1, r * 128:(r + 1) * 128] > 0.0,
                      iota128 + r * 128, _SENT)
            for r in range(_T // 128)
        ]
        return jnp.concatenate(rows, axis=0)

    mip = mask_of(wdpt_ref, 0)
    mis = mask_of(wdst_ref, 4)

    outp, outs = [], []
    c0p = c0s = None
    for _ in range(_C):
        curp = jnp.min(jnp.min(mip, axis=1, keepdims=True), axis=0,
                       keepdims=True)
        curs = jnp.min(jnp.min(mis, axis=1, keepdims=True), axis=0,
                       keepdims=True)
        if c0p is None:
            c0p, c0s = curp, curs
        outp.append(jnp.where(curp < _SENT, curp, c0p))
        outs.append(jnp.where(curs < _SENT, curs, c0s))
        mip = jnp.where(mip == curp, _SENT, mip)
        mis = jnp.where(mis == curs, _SENT, mis)
    cp = jnp.concatenate(outp, axis=1)
    cs = jnp.concatenate(outs, axis=1)
    idx8_ref[...] = jnp.concatenate(
        [cp, cp, cp, cp, cs, cs, cs, cp], axis=0)


def _stage_a(emb, embt, wq4, wtab, wdpt, wdst):
    sd = jax.ShapeDtypeStruct
    return pl.pallas_call(
        _a_body,
        out_shape=[
            sd((_NTAB, _T), jnp.float32),
            sd((_NTAB, _C), jnp.int32),
            sd((_T, 4), jnp.float32),
        ],
    )(emb, embt, wq4, wtab, wdpt, wdst)


def _b_body(tab_hbm, idx_hbm, out_hbm, tab_v, idx_v, out_v):
    wid = lax.axis_index("s") * _NC + lax.axis_index("c")
    tbase = (wid // 4) * _T
    pltpu.sync_copy(tab_hbm.at[pl.ds(tbase, _T)], tab_v)
    pltpu.sync_copy(idx_hbm.at[pl.ds(wid * _L, _L)], idx_v)
    out_v[...] = plsc.load_gather(tab_v, [idx_v[...]])
    pltpu.sync_copy(out_v, out_hbm.at[pl.ds(wid * _L, _L)])


@functools.cache
def _stage_b():
    return functools.partial(
        pl.kernel,
        mesh=plsc.VectorSubcoreMesh(core_axis_name="c", subcore_axis_name="s"),
        compiler_params=pltpu.CompilerParams(
            use_tc_tiling_on_sc=False, needs_layout_passes=False),
        out_type=jax.ShapeDtypeStruct((_NTAB * _C,), jnp.float32),
        scratch_types=[
            pltpu.VMEM((_T,), jnp.float32),
            pltpu.VMEM((_L,), jnp.int32),
            pltpu.VMEM((_L,), jnp.float32),
        ],
    )(_b_body)


def _c_body(qw_ref, g_ref, fo_ref, fa_ref, fv_ref):
    iota = lax.broadcasted_iota(jnp.int32, (_T, _C), 1)

    def head(qcol, krow):
        q2 = qw_ref[:, qcol:qcol + 2]
        kc = g_ref[krow:krow + 2, :]
        s = jnp.dot(q2, kc)
        m = jnp.max(s, axis=1, keepdims=True)
        pos = jnp.min(jnp.where(s == m, iota, _C), axis=1, keepdims=True)
        return iota == pos

    def pick(oh, prow):
        return jnp.sum(jnp.where(oh, g_ref[prow:prow + 1, :], 0.0),
                       axis=1, keepdims=True)

    ohp = head(0, 0)
    fo_ref[...] = pick(ohp, 2)
    fa_ref[...] = pick(ohp, 3)
    ohs = head(2, 4)
    fv_ref[...] = pick(ohs, 6)


def _stage_c(qw, g):
    sd = jax.ShapeDtypeStruct
    return pl.pallas_call(
        _c_body,
        out_shape=[sd((_T, 1), jnp.float32)] * 3,
    )(qw, g)


def kernel(embeddings, WQ_prog, WK_prog, WV_op, WV_arg, WQ_stack, WK_stack, WV_stack):
    dirs = jnp.asarray(_DIRS)
    wdpt = (WK_prog.T @ dirs).T
    wdst = (WK_stack.T @ dirs).T
    wq4 = jnp.concatenate([WQ_prog.T, WQ_stack.T], axis=1)
    wtab = jnp.concatenate([WK_prog, WV_op, WV_arg, WK_stack, WV_stack,
                            WK_prog[0:1]], axis=0)

    tabs, idx8, qw = _stage_a(embeddings, embeddings.T, wq4, wtab, wdpt, wdst)
    out_all = _stage_b()(tabs.reshape(_NTAB * _T), idx8.reshape(_NTAB * _C))
    fo, fa, fv = _stage_c(qw, out_all.reshape(_NTAB, _C))
    return (fo.reshape(_T), fa.reshape(_T), fv.reshape(_T))
